# baseline (speedup 1.0000x reference)
import functools

import jax
import jax.numpy as jnp
from jax.experimental import pallas as pl
from jax.experimental.pallas import tpu as pltpu


def _layer_norm(h, g, b, eps=1e-5):
    mu = jnp.mean(h, axis=-1, keepdims=True)
    var = jnp.mean((h - mu) ** 2, axis=-1, keepdims=True)
    return (h - mu) / jnp.sqrt(var + eps) * g + b


def _gelu_exact(h):
    return 0.5 * h * (1.0 + jax.lax.erf(h * (2.0 ** -0.5)))


def _gate_kernel(x_ref, w1_ref, b1_ref, g1_ref, be1_ref, w2_ref, b2_ref,
                 g2_ref, be2_ref, w3_ref, b3_ref, t_ref,
                 gates_ref, idx_ref, logits_ref):
    dn = (((1,), (1,)), ((), ()))
    x = x_ref[...]
    h = jax.lax.dot_general(x, w1_ref[...], dn, preferred_element_type=jnp.float32) + b1_ref[...]
    h = _gelu_exact(_layer_norm(h, g1_ref[...], be1_ref[...]))
    h = jax.lax.dot_general(h, w2_ref[...], dn, preferred_element_type=jnp.float32) + b2_ref[...]
    h = _gelu_exact(_layer_norm(h, g2_ref[...], be2_ref[...]))
    logits = jax.lax.dot_general(h, w3_ref[...], dn, preferred_element_type=jnp.float32) + b3_ref[...]
    t = jnp.maximum(t_ref[0, 0], 0.1)
    logits = logits / t

    B, E = logits.shape
    col = jax.lax.broadcasted_iota(jnp.int32, (B, E), 1).astype(jnp.float32)
    rev = (E - 1.0) - col
    m1 = jnp.max(logits, axis=-1, keepdims=True)
    a1 = jnp.max(jnp.where(logits == m1, rev, -1.0), axis=-1, keepdims=True)
    i1 = (E - 1.0) - a1
    masked = jnp.where(col == i1, -jnp.inf, logits)
    m2 = jnp.max(masked, axis=-1, keepdims=True)
    a2 = jnp.max(jnp.where(masked == m2, rev, -1.0), axis=-1, keepdims=True)
    i2 = (E - 1.0) - a2

    e2 = jnp.exp(m2 - m1)
    denom = 1.0 + e2
    g1 = 1.0 / denom
    g2 = e2 / denom
    s = g1 + g2 + 1e-8
    g1 = g1 / s
    g2 = g2 / s

    gates_ref[...] = jnp.where(col == i1, g1, 0.0) + jnp.where(col == i2, g2, 0.0)
    logits_ref[...] = logits
    idxcol = jax.lax.broadcasted_iota(jnp.int32, idx_ref.shape, 1)
    idx_ref[...] = jnp.where(idxcol == 0, i1, i2).astype(jnp.int32)


def kernel(x, W1, b1, ln1_g, ln1_b, W2, b2, ln2_g, ln2_b, W3, b3, temperature):
    N, D = x.shape
    H = W1.shape[0]
    H2 = W2.shape[0]
    E = W3.shape[0]
    B = min(4096, N)
    grid = (N // B,)
    IPAD = 2

    row_spec = lambda shape: pl.BlockSpec(shape, lambda i: (i, 0))
    full_spec = lambda shape: pl.BlockSpec(shape, lambda i: (0, 0))

    gates, idx_pad, logits = pl.pallas_call(
        _gate_kernel,
        grid=grid,
        in_specs=[
            row_spec((B, D)),
            full_spec((H, D)), full_spec((1, H)), full_spec((1, H)), full_spec((1, H)),
            full_spec((H2, H)), full_spec((1, H2)), full_spec((1, H2)), full_spec((1, H2)),
            full_spec((E, H2)), full_spec((1, E)),
            pl.BlockSpec(memory_space=pltpu.SMEM),
        ],
        out_specs=[
            row_spec((B, E)),
            row_spec((B, IPAD)),
            row_spec((B, E)),
        ],
        out_shape=[
            jax.ShapeDtypeStruct((N, E), jnp.float32),
            jax.ShapeDtypeStruct((N, IPAD), jnp.int32),
            jax.ShapeDtypeStruct((N, E), jnp.float32),
        ],
    )(
        x,
        W1, b1.reshape(1, H), ln1_g.reshape(1, H), ln1_b.reshape(1, H),
        W2, b2.reshape(1, H2), ln2_g.reshape(1, H2), ln2_b.reshape(1, H2),
        W3, b3.reshape(1, E),
        temperature.reshape(1, 1),
    )
    return gates, idx_pad, logits


# speedup vs baseline: 1.0009x; 1.0009x over previous
import functools

import jax
import jax.numpy as jnp
from jax.experimental import pallas as pl
from jax.experimental.pallas import tpu as pltpu


def _layer_norm(h, g, b, eps=1e-5):
    mu = jnp.mean(h, axis=-1, keepdims=True)
    var = jnp.mean((h - mu) ** 2, axis=-1, keepdims=True)
    return (h - mu) / jnp.sqrt(var + eps) * g + b


def _gelu_exact(h):
    return 0.5 * h * (1.0 + jax.lax.erf(h * (2.0 ** -0.5)))


def _gate_kernel(x_ref, w1_ref, b1_ref, g1_ref, be1_ref, w2_ref, b2_ref,
                 g2_ref, be2_ref, w3_ref, b3_ref, t_ref,
                 gates_ref, idx_ref, logits_ref):
    dn = (((1,), (1,)), ((), ()))
    x = x_ref[...]
    h = jax.lax.dot_general(x, w1_ref[...], dn, preferred_element_type=jnp.float32) + b1_ref[...]
    h = _gelu_exact(_layer_norm(h, g1_ref[...], be1_ref[...]))
    h = jax.lax.dot_general(h, w2_ref[...], dn, preferred_element_type=jnp.float32) + b2_ref[...]
    h = _gelu_exact(_layer_norm(h, g2_ref[...], be2_ref[...]))
    logits = jax.lax.dot_general(h, w3_ref[...], dn, preferred_element_type=jnp.float32) + b3_ref[...]
    t = jnp.maximum(t_ref[0, 0], 0.1)
    logits = logits / t

    B, E = logits.shape
    col = jax.lax.broadcasted_iota(jnp.int32, (B, E), 1).astype(jnp.float32)
    rev = (E - 1.0) - col
    m1 = jnp.max(logits, axis=-1, keepdims=True)
    a1 = jnp.max(jnp.where(logits == m1, rev, -1.0), axis=-1, keepdims=True)
    i1 = (E - 1.0) - a1
    masked = jnp.where(col == i1, -jnp.inf, logits)
    m2 = jnp.max(masked, axis=-1, keepdims=True)
    a2 = jnp.max(jnp.where(masked == m2, rev, -1.0), axis=-1, keepdims=True)
    i2 = (E - 1.0) - a2

    e2 = jnp.exp(m2 - m1)
    denom = 1.0 + e2
    g1 = 1.0 / denom
    g2 = e2 / denom
    s = g1 + g2 + 1e-8
    g1 = g1 / s
    g2 = g2 / s

    gates_ref[...] = jnp.where(col == i1, g1, 0.0) + jnp.where(col == i2, g2, 0.0)
    logits_ref[...] = logits
    idxcol = jax.lax.broadcasted_iota(jnp.int32, idx_ref.shape, 1)
    idx_ref[...] = jnp.where(idxcol == 0, i1, i2).astype(jnp.int32)


def kernel(x, W1, b1, ln1_g, ln1_b, W2, b2, ln2_g, ln2_b, W3, b3, temperature):
    N, D = x.shape
    H = W1.shape[0]
    H2 = W2.shape[0]
    E = W3.shape[0]
    B = min(4096, N)
    grid = (N // B,)
    IPAD = 2

    row_spec = lambda shape: pl.BlockSpec(shape, lambda i: (i, 0))
    full_spec = lambda shape: pl.BlockSpec(shape, lambda i: (0, 0))

    gates, idx_pad, logits = pl.pallas_call(
        _gate_kernel,
        grid=grid,
        compiler_params=pltpu.CompilerParams(
            dimension_semantics=("parallel",)),
        in_specs=[
            row_spec((B, D)),
            full_spec((H, D)), full_spec((1, H)), full_spec((1, H)), full_spec((1, H)),
            full_spec((H2, H)), full_spec((1, H2)), full_spec((1, H2)), full_spec((1, H2)),
            full_spec((E, H2)), full_spec((1, E)),
            pl.BlockSpec(memory_space=pltpu.SMEM),
        ],
        out_specs=[
            row_spec((B, E)),
            row_spec((B, IPAD)),
            row_spec((B, E)),
        ],
        out_shape=[
            jax.ShapeDtypeStruct((N, E), jnp.float32),
            jax.ShapeDtypeStruct((N, IPAD), jnp.int32),
            jax.ShapeDtypeStruct((N, E), jnp.float32),
        ],
    )(
        x,
        W1, b1.reshape(1, H), ln1_g.reshape(1, H), ln1_b.reshape(1, H),
        W2, b2.reshape(1, H2), ln2_g.reshape(1, H2), ln2_b.reshape(1, H2),
        W3, b3.reshape(1, E),
        temperature.reshape(1, 1),
    )
    return gates, idx_pad, logits
